# Initial kernel scaffold; baseline (speedup 1.0000x reference)
import jax
import jax.numpy as jnp
from jax.experimental import pallas as pl


def kernel(x, mask):
    raise NotImplementedError("write your pallas kernel here")


# speedup vs baseline: 1.0696x; 1.0696x over previous
import jax
import jax.numpy as jnp
from jax.experimental import pallas as pl
from jax.experimental.pallas import tpu as pltpu


def _rev_blend_kernel(mask_ref, x_ref, o_ref):
    a = x_ref[...]
    c_blk, hw = a.shape
    half = hw // 2
    idx = half - 1 - jax.lax.broadcasted_iota(jnp.int32, (c_blk, half), 1)
    r0 = jnp.take_along_axis(a[:, half:], idx, axis=1)
    r1 = jnp.take_along_axis(a[:, :half], idx, axis=1)
    r = jnp.concatenate([r0, r1], axis=1)
    keep = mask_ref[...] != 0
    o_ref[...] = jnp.where(keep, a, r)


@jax.jit
def kernel(x, mask):
    B, C, H, W = x.shape
    n = B * C
    hw = H * W
    xf = x.reshape(n, hw)
    mf = jnp.repeat(mask, C).reshape(n, 1)

    c_blk = 1024 if n % 1024 == 0 else n
    grid = (n // c_blk,)
    out = pl.pallas_call(
        _rev_blend_kernel,
        out_shape=jax.ShapeDtypeStruct((n, hw), x.dtype),
        grid=grid,
        in_specs=[pl.BlockSpec((c_blk, 1), lambda i: (i, 0)),
                  pl.BlockSpec((c_blk, hw), lambda i: (i, 0))],
        out_specs=pl.BlockSpec((c_blk, hw), lambda i: (i, 0)),
        compiler_params=pltpu.CompilerParams(
            dimension_semantics=("parallel",)),
    )(mf, xf)
    return out.reshape(B, C, H, W)


# speedup vs baseline: 7.0517x; 6.5928x over previous
import jax
import jax.numpy as jnp
from jax.experimental import pallas as pl
from jax.experimental.pallas import tpu as pltpu


def _rev_s_blend_kernel(mask_ref, x_ref, o_ref):
    a = x_ref[...]
    s = a.shape[1]
    parts = []
    for t in range(s // 8 - 1, -1, -1):
        tile = a[:, t * 8:(t + 1) * 8, :]
        parts.extend(tile[:, r:r + 1, :] for r in range(7, -1, -1))
    r = jnp.concatenate(parts, axis=1)
    keep = (mask_ref[...] != 0)[:, :, None]
    o_ref[...] = jnp.where(keep, a, r)


@jax.jit
def kernel(x, mask):
    B, C, H, W = x.shape
    S = H * W
    xs = jnp.transpose(x, (0, 2, 3, 1)).reshape(B, S, C)
    m2 = mask.reshape(B, 1)

    bb = 8 if B % 8 == 0 else B
    grid = (B // bb,)
    out = pl.pallas_call(
        _rev_s_blend_kernel,
        out_shape=jax.ShapeDtypeStruct((B, S, C), x.dtype),
        grid=grid,
        in_specs=[pl.BlockSpec((bb, 1), lambda i: (i, 0)),
                  pl.BlockSpec((bb, S, C), lambda i: (i, 0, 0))],
        out_specs=pl.BlockSpec((bb, S, C), lambda i: (i, 0, 0)),
        compiler_params=pltpu.CompilerParams(
            dimension_semantics=("parallel",)),
    )(m2, xs)
    return jnp.transpose(out.reshape(B, H, W, C), (0, 3, 1, 2))


# speedup vs baseline: 7.9075x; 1.1214x over previous
import jax
import jax.numpy as jnp
from jax.experimental import pallas as pl
from jax.experimental.pallas import tpu as pltpu


def _rev_s_blend_kernel(mask_ref, x_ref, o_ref):
    a = x_ref[...]
    s = a.shape[1]
    parts = []
    for t in range(s // 8 - 1, -1, -1):
        tile = a[:, t * 8:(t + 1) * 8, :]
        parts.extend(tile[:, r:r + 1, :] for r in range(7, -1, -1))
    r = jnp.concatenate(parts, axis=1)
    keep = (mask_ref[...] != 0)[:, :, None]
    o_ref[...] = jnp.where(keep, a, r)


@jax.jit
def kernel(x, mask):
    B, C, H, W = x.shape
    S = H * W
    xs = jnp.transpose(x, (0, 2, 3, 1)).reshape(B, S, C)
    m2 = mask.reshape(B, 1)

    bb = 16 if B % 16 == 0 else B
    grid = (B // bb,)
    out = pl.pallas_call(
        _rev_s_blend_kernel,
        out_shape=jax.ShapeDtypeStruct((B, S, C), x.dtype),
        grid=grid,
        in_specs=[pl.BlockSpec((bb, 1), lambda i: (i, 0)),
                  pl.BlockSpec((bb, S, C), lambda i: (i, 0, 0))],
        out_specs=pl.BlockSpec((bb, S, C), lambda i: (i, 0, 0)),
        compiler_params=pltpu.CompilerParams(
            dimension_semantics=("parallel",)),
    )(m2, xs)
    return jnp.transpose(out.reshape(B, H, W, C), (0, 3, 1, 2))
